# baseline (speedup 1.0000x reference)
import jax
import jax.numpy as jnp
from jax.experimental import pallas as pl
from jax.experimental.pallas import tpu as pltpu

G = 1024
NPG = 48
EPG = 192
E = G * EPG
DF = 128
NH = 128
K1, K2, K3 = 24, 12, 6
EDDI = 8192
BS = 4096
DDIH = 128
DE = 16

BG = 128
NBLK = G // BG
DCH = 1024


def _bmm(a, b):
    return jax.lax.dot_general(a, b, (((2,), (1,)), ((0,), (0,))),
                               preferred_element_type=jnp.float32)


def _col_to_row(v):
    bsz, n, _ = v.shape
    i1 = jax.lax.broadcasted_iota(jnp.int32, (bsz, n, n), 1)
    i2 = jax.lax.broadcasted_iota(jnp.int32, (bsz, n, n), 2)
    eye = (i1 == i2).astype(jnp.float32)
    return jnp.sum(eye * v, axis=1, keepdims=True)


def _layer(h, St, Dt, w_row, W, brow, wr_row, wn_row, bp, npg, k):
    bsz = h.shape[0]
    hW = (h.reshape(bsz * npg, h.shape[2]) @ W).reshape(bsz, npg, NH)
    Araw = jax.lax.dot_general(Dt * w_row, St,
                               (((2,), (2,)), ((0,), (0,))),
                               preferred_element_type=jnp.float32)
    deg = jnp.sum(Araw, axis=2, keepdims=True) + 1.0
    dis = jax.lax.rsqrt(deg)
    ii = jax.lax.broadcasted_iota(jnp.int32, (bsz, npg, npg), 1)
    jj = jax.lax.broadcasted_iota(jnp.int32, (bsz, npg, npg), 2)
    eye = (ii == jj).astype(jnp.float32)
    out = dis * _bmm(Araw + eye, dis * hW) + brow
    hh = jnp.maximum(out, 0.0)
    g = _bmm(Araw, hh)
    s = jnp.sum(hh * wr_row + g * wn_row, axis=2, keepdims=True) + bp
    s_row = _col_to_row(s)
    beats = ((s_row > s) | ((s_row == s) & (jj < ii))).astype(jnp.float32)
    rank_row = (npg - 1.0) - jnp.sum(beats, axis=1, keepdims=True)
    rr = jax.lax.broadcasted_iota(jnp.int32, (bsz, k, npg), 1).astype(jnp.float32)
    P = (rank_row == rr).astype(jnp.float32)
    hp = _bmm(P, hh * jnp.tanh(s))
    S2t = _bmm(P, St)
    D2t = _bmm(P, Dt)
    return hp, S2t, D2t


def _gnn_step(x_ref, sl_ref, dl_ref, w_ref, wg_refs, feat):
    (W1_ref, b1_ref, wr1_ref, wn1_ref, bp1_ref,
     W2_ref, b2_ref, wr2_ref, wn2_ref, bp2_ref,
     W3_ref, b3_ref, wr3_ref, wn3_ref, bp3_ref) = wg_refs
    bsz = BG
    x3 = x_ref[...].reshape(bsz, NPG, DF)
    sl = sl_ref[...]
    dl = dl_ref[...]
    w = w_ref[...]
    vv = jax.lax.broadcasted_iota(jnp.int32, (bsz, NPG, EPG), 1)
    S1 = (sl == vv).astype(jnp.float32)
    D1 = (dl == vv).astype(jnp.float32)

    def rowify(r):
        return r[...].reshape(1, 1, NH)

    hp1, S2, D2 = _layer(x3, S1, D1, w, W1_ref[...],
                         rowify(b1_ref), rowify(wr1_ref), rowify(wn1_ref),
                         bp1_ref[0, 0], NPG, K1)
    hp2, S3, D3 = _layer(hp1, S2, D2, w, W2_ref[...],
                         rowify(b2_ref), rowify(wr2_ref), rowify(wn2_ref),
                         bp2_ref[0, 0], K1, K2)
    hp3, _, _ = _layer(hp2, S3, D3, w, W3_ref[...],
                       rowify(b3_ref), rowify(wr3_ref), rowify(wn3_ref),
                       bp3_ref[0, 0], K2, K3)
    off = jnp.minimum(pl.program_id(0), NBLK - 1) * BG
    feat[pl.ds(off, BG), 0 * NH:1 * NH] = jnp.max(hp1, axis=1)
    feat[pl.ds(off, BG), 1 * NH:2 * NH] = jnp.mean(hp1, axis=1)
    feat[pl.ds(off, BG), 2 * NH:3 * NH] = jnp.max(hp2, axis=1)
    feat[pl.ds(off, BG), 3 * NH:4 * NH] = jnp.mean(hp2, axis=1)
    feat[pl.ds(off, BG), 4 * NH:5 * NH] = jnp.max(hp3, axis=1)
    feat[pl.ds(off, BG), 5 * NH:6 * NH] = jnp.mean(hp3, axis=1)


def _ddi_step(feat_ref, dsrc_ref, ddst_ref, attr_ref, wd_refs,
              loss_ref, np_ref, nn_ref, pfx_ref):
    (Wd_ref, bd_ref, Wl1_ref, bl1_ref, Wl2_ref, bl2_ref,
     Wl3_ref, bl3_ref) = wd_refs
    feat = feat_ref[...]
    hW = feat @ Wd_ref[...]
    nio = jax.lax.broadcasted_iota(jnp.int32, (DCH, G), 1)
    nch = EDDI // DCH

    deg = jnp.zeros((G, 1), jnp.float32)
    ones_col = jnp.ones((DCH, 1), jnp.bfloat16)
    for c in range(nch):
        dc = ddst_ref[c * DCH:(c + 1) * DCH, :]
        Dc = (dc == nio).astype(jnp.bfloat16)
        deg = deg + jax.lax.dot_general(
            Dc, ones_col, (((0,), (0,)), ((), ())),
            preferred_element_type=jnp.float32)
    deg = deg + 1.0
    dis = jax.lax.rsqrt(deg)

    xh = (dis * hW).astype(jnp.bfloat16)
    msg = jnp.zeros((G, DDIH), jnp.float32)
    for c in range(nch):
        sc = dsrc_ref[c * DCH:(c + 1) * DCH, :]
        dc = ddst_ref[c * DCH:(c + 1) * DCH, :]
        Sc = (sc == nio).astype(jnp.bfloat16)
        Dc = (dc == nio).astype(jnp.bfloat16)
        hsrc = jax.lax.dot_general(Sc, xh, (((1,), (0,)), ((), ())),
                                   preferred_element_type=jnp.float32)
        msg = msg + jax.lax.dot_general(
            Dc, hsrc.astype(jnp.bfloat16), (((0,), (0,)), ((), ())),
            preferred_element_type=jnp.float32)
    xd = jnp.maximum(dis * msg + (dis * dis) * hW + bd_ref[...], 0.0)

    fxa = (xd @ Wl1_ref[...]).astype(jnp.bfloat16)
    fya = (xd @ Wl2_ref[...]).astype(jnp.bfloat16)
    bl1 = bl1_ref[...]
    bl2 = bl2_ref[...]
    Wl3 = Wl3_ref[...]
    bl3 = bl3_ref[...]
    for c in range(nch):
        sc = dsrc_ref[c * DCH:(c + 1) * DCH, :]
        dc = ddst_ref[c * DCH:(c + 1) * DCH, :]
        Sc = (sc == nio).astype(jnp.bfloat16)
        Dc = (dc == nio).astype(jnp.bfloat16)
        fx = jax.nn.sigmoid(jax.lax.dot_general(
            Sc, fxa, (((1,), (0,)), ((), ())),
            preferred_element_type=jnp.float32) + bl1)
        fy = jax.nn.sigmoid(jax.lax.dot_general(
            Dc, fya, (((1,), (0,)), ((), ())),
            preferred_element_type=jnp.float32) + bl2)
        fa = jax.nn.sigmoid(attr_ref[c * DCH:(c + 1) * DCH, :] @ Wl3 + bl3)
        lv = fx + fa - fy
        nrm = jnp.sqrt(jnp.sum(lv * lv, axis=1, keepdims=True))
        r = (c % (BS // DCH)) * DCH
        if c < BS // DCH:
            np_ref[r:r + DCH, :] = nrm
            pfx_ref[r:r + DCH, :] = fx
        else:
            nn_ref[r:r + DCH, :] = nrm
    loss_ref[...] = (2.0 * DDIH - np_ref[...]) + nn_ref[...]


def _fused(x_ref, sl_ref, dl_ref, w_ref,
           W1_ref, b1_ref, wr1_ref, wn1_ref, bp1_ref,
           W2_ref, b2_ref, wr2_ref, wn2_ref, bp2_ref,
           W3_ref, b3_ref, wr3_ref, wn3_ref, bp3_ref,
           dsrc_ref, ddst_ref, attr_ref,
           Wd_ref, bd_ref, Wl1_ref, bl1_ref, Wl2_ref, bl2_ref,
           Wl3_ref, bl3_ref,
           loss_ref, np_ref, nn_ref, pfx_ref, feat):
    i = pl.program_id(0)

    @pl.when(i < NBLK)
    def _gnn():
        _gnn_step(x_ref, sl_ref, dl_ref, w_ref,
                  (W1_ref, b1_ref, wr1_ref, wn1_ref, bp1_ref,
                   W2_ref, b2_ref, wr2_ref, wn2_ref, bp2_ref,
                   W3_ref, b3_ref, wr3_ref, wn3_ref, bp3_ref), feat)

    @pl.when(i == NBLK)
    def _ddi():
        _ddi_step(feat, dsrc_ref, ddst_ref, attr_ref,
                  (Wd_ref, bd_ref, Wl1_ref, bl1_ref, Wl2_ref, bl2_ref,
                   Wl3_ref, bl3_ref),
                  loss_ref, np_ref, nn_ref, pfx_ref)


def kernel(x, edge_index, edge_weight, batch, ddi_edge_index, ddi_edge_attr,
           W1, b1, Wp1r, Wp1n, bp1, W2, b2, Wp2r, Wp2n, bp2,
           W3, b3, Wp3r, Wp3n, bp3,
           Wd, bd, Wl1, bl1, Wl2, bl2, Wl3, bl3):
    ei = edge_index.astype(jnp.int32)
    sl = (ei[0] % NPG).reshape(G, 1, EPG)
    dl = (ei[1] % NPG).reshape(G, 1, EPG)
    w3 = edge_weight.reshape(G, 1, EPG)
    di = ddi_edge_index.astype(jnp.int32)
    dsrc = di[0].reshape(EDDI, 1)
    ddst = di[1].reshape(EDDI, 1)

    def row(a):
        return a.reshape(1, -1)

    def blk(i):
        return jnp.minimum(i, NBLK - 1)

    fixed = lambda i: (0, 0)
    gw_specs = [
        pl.BlockSpec((DF, NH), fixed), pl.BlockSpec((1, NH), fixed),
        pl.BlockSpec((1, NH), fixed), pl.BlockSpec((1, NH), fixed),
        pl.BlockSpec((1, 1), fixed),
        pl.BlockSpec((NH, NH), fixed), pl.BlockSpec((1, NH), fixed),
        pl.BlockSpec((1, NH), fixed), pl.BlockSpec((1, NH), fixed),
        pl.BlockSpec((1, 1), fixed),
        pl.BlockSpec((NH, NH), fixed), pl.BlockSpec((1, NH), fixed),
        pl.BlockSpec((1, NH), fixed), pl.BlockSpec((1, NH), fixed),
        pl.BlockSpec((1, 1), fixed),
    ]
    ddi_specs = [
        pl.BlockSpec((EDDI, 1), fixed),
        pl.BlockSpec((EDDI, 1), fixed),
        pl.BlockSpec((EDDI, DE), fixed),
        pl.BlockSpec((6 * NH, DDIH), fixed), pl.BlockSpec((1, DDIH), fixed),
        pl.BlockSpec((DDIH, DDIH), fixed), pl.BlockSpec((1, DDIH), fixed),
        pl.BlockSpec((DDIH, DDIH), fixed), pl.BlockSpec((1, DDIH), fixed),
        pl.BlockSpec((DE, DDIH), fixed), pl.BlockSpec((1, DDIH), fixed),
    ]
    loss2, np2, nn2, pfx = pl.pallas_call(
        _fused,
        grid=(NBLK + 1,),
        in_specs=[
            pl.BlockSpec((BG * NPG, DF), lambda i: (blk(i), 0)),
            pl.BlockSpec((BG, 1, EPG), lambda i: (blk(i), 0, 0)),
            pl.BlockSpec((BG, 1, EPG), lambda i: (blk(i), 0, 0)),
            pl.BlockSpec((BG, 1, EPG), lambda i: (blk(i), 0, 0)),
        ] + gw_specs + ddi_specs,
        out_specs=(
            pl.BlockSpec((BS, 1), fixed),
            pl.BlockSpec((BS, 1), fixed),
            pl.BlockSpec((BS, 1), fixed),
            pl.BlockSpec((BS, DDIH), fixed),
        ),
        out_shape=(
            jax.ShapeDtypeStruct((BS, 1), jnp.float32),
            jax.ShapeDtypeStruct((BS, 1), jnp.float32),
            jax.ShapeDtypeStruct((BS, 1), jnp.float32),
            jax.ShapeDtypeStruct((BS, DDIH), jnp.float32),
        ),
        scratch_shapes=[pltpu.VMEM((G, 6 * NH), jnp.float32)],
    )(x, sl, dl, w3,
      W1, row(b1), Wp1r.reshape(1, NH), Wp1n.reshape(1, NH), bp1.reshape(1, 1),
      W2, row(b2), Wp2r.reshape(1, NH), Wp2n.reshape(1, NH), bp2.reshape(1, 1),
      W3, row(b3), Wp3r.reshape(1, NH), Wp3n.reshape(1, NH), bp3.reshape(1, 1),
      dsrc, ddst, ddi_edge_attr,
      Wd, row(bd), Wl1, row(bl1), Wl2, row(bl2), Wl3, row(bl3))

    return (loss2.reshape(BS), np2.reshape(BS), nn2.reshape(BS), pfx)


# speedup vs baseline: 1.1629x; 1.1629x over previous
import jax
import jax.numpy as jnp
from jax.experimental import pallas as pl

G = 1024
NPG = 48
EPG = 192
E = G * EPG
DF = 128
NH = 128
K1, K2, K3 = 24, 12, 6
EDDI = 8192
BS = 4096
DDIH = 128
DE = 16

BG = 128
DCH = 1024


def _bmm(a, b):
    return jax.lax.dot_general(a, b, (((2,), (1,)), ((0,), (0,))),
                               preferred_element_type=jnp.float32)


def _col_to_row(v):
    return jnp.swapaxes(v, 1, 2)


def _layer(h, St, Dt, w_row, W, brow, wr_row, wn_row, bp, npg, k, last=False):
    bsz = h.shape[0]
    hW = (h.reshape(bsz * npg, h.shape[2]) @ W).reshape(bsz, npg, NH)
    Araw = jax.lax.dot_general(Dt * w_row, St,
                               (((2,), (2,)), ((0,), (0,))),
                               preferred_element_type=jnp.float32)
    deg = jnp.sum(Araw, axis=2, keepdims=True) + 1.0
    dis = jax.lax.rsqrt(deg)
    ii = jax.lax.broadcasted_iota(jnp.int32, (bsz, npg, npg), 1)
    jj = jax.lax.broadcasted_iota(jnp.int32, (bsz, npg, npg), 2)
    eye = (ii == jj).astype(jnp.float32)
    out = dis * _bmm(Araw + eye, dis * hW) + brow
    hh = jnp.maximum(out, 0.0)
    g = _bmm(Araw, hh)
    s = jnp.sum(hh * wr_row + g * wn_row, axis=2, keepdims=True) + bp
    s_row = _col_to_row(s)
    beats = ((s_row > s) | ((s_row == s) & (jj < ii))).astype(jnp.float32)
    rank_row = (npg - 1.0) - jnp.sum(beats, axis=1, keepdims=True)
    rr = jax.lax.broadcasted_iota(jnp.int32, (bsz, k, npg), 1).astype(jnp.float32)
    P = (rank_row == rr).astype(jnp.float32)
    hp = _bmm(P, hh * jnp.tanh(s))
    if last:
        return hp, None, None, w_row
    S2t = _bmm(P, St)
    D2t = _bmm(P, Dt)
    return hp, S2t, D2t, w_row


def _gnn_block(x_ref, sl_ref, dl_ref, w_ref,
               W1_ref, b1_ref, wr1_ref, wn1_ref, bp1_ref,
               W2_ref, b2_ref, wr2_ref, wn2_ref, bp2_ref,
               W3_ref, b3_ref, wr3_ref, wn3_ref, bp3_ref,
               out_ref):
    bsz = BG
    x3 = x_ref[...].reshape(bsz, NPG, DF)
    sl = sl_ref[...]
    dl = dl_ref[...]
    w = w_ref[...]
    vv = jax.lax.broadcasted_iota(jnp.int32, (bsz, NPG, EPG), 1)
    S1 = (sl == vv).astype(jnp.float32)
    D1 = (dl == vv).astype(jnp.float32)

    def rowify(r):
        return r[...].reshape(1, 1, NH)

    hp1, S2, D2, w2 = _layer(x3, S1, D1, w, W1_ref[...],
                             rowify(b1_ref), rowify(wr1_ref), rowify(wn1_ref),
                             bp1_ref[0, 0], NPG, K1)
    hp2, S3, D3, w3 = _layer(hp1, S2, D2, w2, W2_ref[...],
                             rowify(b2_ref), rowify(wr2_ref), rowify(wn2_ref),
                             bp2_ref[0, 0], K1, K2)
    hp3, _, _, _ = _layer(hp2, S3, D3, w3, W3_ref[...],
                          rowify(b3_ref), rowify(wr3_ref), rowify(wn3_ref),
                          bp3_ref[0, 0], K2, K3, last=True)
    out_ref[:, 0 * NH:1 * NH] = jnp.max(hp1, axis=1)
    out_ref[:, 1 * NH:2 * NH] = jnp.mean(hp1, axis=1)
    out_ref[:, 2 * NH:3 * NH] = jnp.max(hp2, axis=1)
    out_ref[:, 3 * NH:4 * NH] = jnp.mean(hp2, axis=1)
    out_ref[:, 4 * NH:5 * NH] = jnp.max(hp3, axis=1)
    out_ref[:, 5 * NH:6 * NH] = jnp.mean(hp3, axis=1)


def _ddi_block(feat_ref, dsrc_ref, ddst_ref, attr_ref,
               Wd_ref, bd_ref, Wl1_ref, bl1_ref, Wl2_ref, bl2_ref,
               Wl3_ref, bl3_ref,
               loss_ref, np_ref, nn_ref, pfx_ref):
    feat = feat_ref[...]
    hW = feat @ Wd_ref[...]
    nio = jax.lax.broadcasted_iota(jnp.int32, (DCH, G), 1)
    nch = EDDI // DCH

    deg = jnp.zeros((G, 1), jnp.float32)
    ones_col = jnp.ones((DCH, 1), jnp.bfloat16)
    for c in range(nch):
        dc = ddst_ref[c * DCH:(c + 1) * DCH, :]
        Dc = (dc == nio).astype(jnp.bfloat16)
        deg = deg + jax.lax.dot_general(
            Dc, ones_col, (((0,), (0,)), ((), ())),
            preferred_element_type=jnp.float32)
    deg = deg + 1.0
    dis = jax.lax.rsqrt(deg)

    xh = (dis * hW).astype(jnp.bfloat16)
    msg = jnp.zeros((G, DDIH), jnp.float32)
    for c in range(nch):
        sc = dsrc_ref[c * DCH:(c + 1) * DCH, :]
        dc = ddst_ref[c * DCH:(c + 1) * DCH, :]
        Sc = (sc == nio).astype(jnp.bfloat16)
        Dc = (dc == nio).astype(jnp.bfloat16)
        hsrc = jax.lax.dot_general(Sc, xh, (((1,), (0,)), ((), ())),
                                   preferred_element_type=jnp.float32)
        msg = msg + jax.lax.dot_general(
            Dc, hsrc.astype(jnp.bfloat16), (((0,), (0,)), ((), ())),
            preferred_element_type=jnp.float32)
    xd = jnp.maximum(dis * msg + (dis * dis) * hW + bd_ref[...], 0.0)

    fxa = (xd @ Wl1_ref[...]).astype(jnp.bfloat16)
    fya = (xd @ Wl2_ref[...]).astype(jnp.bfloat16)
    bl1 = bl1_ref[...]
    bl2 = bl2_ref[...]
    Wl3 = Wl3_ref[...]
    bl3 = bl3_ref[...]
    for c in range(nch):
        sc = dsrc_ref[c * DCH:(c + 1) * DCH, :]
        dc = ddst_ref[c * DCH:(c + 1) * DCH, :]
        Sc = (sc == nio).astype(jnp.bfloat16)
        Dc = (dc == nio).astype(jnp.bfloat16)
        fx = jax.nn.sigmoid(jax.lax.dot_general(
            Sc, fxa, (((1,), (0,)), ((), ())),
            preferred_element_type=jnp.float32) + bl1)
        fy = jax.nn.sigmoid(jax.lax.dot_general(
            Dc, fya, (((1,), (0,)), ((), ())),
            preferred_element_type=jnp.float32) + bl2)
        fa = jax.nn.sigmoid(attr_ref[c * DCH:(c + 1) * DCH, :] @ Wl3 + bl3)
        lv = fx + fa - fy
        nrm = jnp.sqrt(jnp.sum(lv * lv, axis=1, keepdims=True))
        r = (c % (BS // DCH)) * DCH
        if c < BS // DCH:
            np_ref[r:r + DCH, :] = nrm
            pfx_ref[r:r + DCH, :] = fx
        else:
            nn_ref[r:r + DCH, :] = nrm
    loss_ref[...] = (2.0 * DDIH - np_ref[...]) + nn_ref[...]


def kernel(x, edge_index, edge_weight, batch, ddi_edge_index, ddi_edge_attr,
           W1, b1, Wp1r, Wp1n, bp1, W2, b2, Wp2r, Wp2n, bp2,
           W3, b3, Wp3r, Wp3n, bp3,
           Wd, bd, Wl1, bl1, Wl2, bl2, Wl3, bl3):
    ei = edge_index.astype(jnp.int32)
    sl = (ei[0] % NPG).reshape(G, 1, EPG)
    dl = (ei[1] % NPG).reshape(G, 1, EPG)
    w3 = edge_weight.reshape(G, 1, EPG)

    def row(a):
        return a.reshape(1, -1)

    wspecs = [
        pl.BlockSpec((DF, NH), lambda i: (0, 0)),
        pl.BlockSpec((1, NH), lambda i: (0, 0)),
        pl.BlockSpec((1, NH), lambda i: (0, 0)),
        pl.BlockSpec((1, NH), lambda i: (0, 0)),
        pl.BlockSpec((1, 1), lambda i: (0, 0)),
    ]
    feat = pl.pallas_call(
        _gnn_block,
        grid=(G // BG,),
        in_specs=[
            pl.BlockSpec((BG * NPG, DF), lambda i: (i, 0)),
            pl.BlockSpec((BG, 1, EPG), lambda i: (i, 0, 0)),
            pl.BlockSpec((BG, 1, EPG), lambda i: (i, 0, 0)),
            pl.BlockSpec((BG, 1, EPG), lambda i: (i, 0, 0)),
        ] + wspecs + [
            pl.BlockSpec((NH, NH), lambda i: (0, 0)),
            pl.BlockSpec((1, NH), lambda i: (0, 0)),
            pl.BlockSpec((1, NH), lambda i: (0, 0)),
            pl.BlockSpec((1, NH), lambda i: (0, 0)),
            pl.BlockSpec((1, 1), lambda i: (0, 0)),
            pl.BlockSpec((NH, NH), lambda i: (0, 0)),
            pl.BlockSpec((1, NH), lambda i: (0, 0)),
            pl.BlockSpec((1, NH), lambda i: (0, 0)),
            pl.BlockSpec((1, NH), lambda i: (0, 0)),
            pl.BlockSpec((1, 1), lambda i: (0, 0)),
        ],
        out_specs=pl.BlockSpec((BG, 6 * NH), lambda i: (i, 0)),
        out_shape=jax.ShapeDtypeStruct((G, 6 * NH), jnp.float32),
    )(x, sl, dl, w3,
      W1, row(b1), Wp1r.reshape(1, NH), Wp1n.reshape(1, NH), bp1.reshape(1, 1),
      W2, row(b2), Wp2r.reshape(1, NH), Wp2n.reshape(1, NH), bp2.reshape(1, 1),
      W3, row(b3), Wp3r.reshape(1, NH), Wp3n.reshape(1, NH), bp3.reshape(1, 1))

    di = ddi_edge_index.astype(jnp.int32)
    dsrc = di[0].reshape(EDDI, 1)
    ddst = di[1].reshape(EDDI, 1)
    loss2, np2, nn2, pfx = pl.pallas_call(
        _ddi_block,
        out_shape=(
            jax.ShapeDtypeStruct((BS, 1), jnp.float32),
            jax.ShapeDtypeStruct((BS, 1), jnp.float32),
            jax.ShapeDtypeStruct((BS, 1), jnp.float32),
            jax.ShapeDtypeStruct((BS, DDIH), jnp.float32),
        ),
    )(feat, dsrc, ddst, ddi_edge_attr,
      Wd, row(bd), Wl1, row(bl1), Wl2, row(bl2), Wl3, row(bl3))

    return (loss2.reshape(BS), np2.reshape(BS), nn2.reshape(BS), pfx)
